# Initial kernel scaffold; baseline (speedup 1.0000x reference)
import jax
import jax.numpy as jnp
from jax.experimental import pallas as pl


def kernel(char_ids, table):
    raise NotImplementedError("write your pallas kernel here")


# speedup vs baseline: 3.5359x; 3.5359x over previous
import functools

import jax
import jax.numpy as jnp
from jax import lax
from jax.experimental import pallas as pl
from jax.experimental.pallas import tpu as pltpu
from jax.experimental.pallas import tpu_sc as plsc

EMB = 64
NC = 2
NS = 16
NW = NC * NS
C = 128


@functools.partial(jax.jit, static_argnums=(2,))
def _gather_sc(idx, table, nchunk):
    mesh = plsc.VectorSubcoreMesh(core_axis_name="c", subcore_axis_name="s")

    @functools.partial(
        pl.kernel,
        mesh=mesh,
        out_type=jax.ShapeDtypeStruct((NW, nchunk, C, EMB), jnp.float32),
        scratch_types=[
            pltpu.VMEM((nchunk, C), jnp.int32),
            pltpu.VMEM((C, EMB), jnp.float32),
            pltpu.SemaphoreType.DMA,
        ],
        compiler_params=pltpu.CompilerParams(use_tc_tiling_on_sc=False),
    )
    def k(idx_hbm, table_hbm, out_hbm, idx_v, rows_v, sem):
        wid = lax.axis_index("s") * NC + lax.axis_index("c")
        pltpu.sync_copy(idx_hbm.at[wid], idx_v)

        def step(j, carry):
            pltpu.async_copy(table_hbm.at[idx_v.at[j]], rows_v, sem).wait()
            pltpu.sync_copy(rows_v, out_hbm.at[wid, j])
            return carry

        lax.fori_loop(0, nchunk, step, 0)

    return k(idx, table)


def kernel(char_ids, table):
    B, L = char_ids.shape
    total = B * L
    assert total % (NW * C) == 0
    nchunk = total // (NW * C)
    idx = char_ids.reshape(NW, nchunk, C)
    out = _gather_sc(idx, table, nchunk)
    return out.reshape(B, L, EMB)


# speedup vs baseline: 4.2407x; 1.1993x over previous
import functools

import jax
import jax.numpy as jnp
from jax import lax
from jax.experimental import pallas as pl
from jax.experimental.pallas import tpu as pltpu
from jax.experimental.pallas import tpu_sc as plsc

EMB = 64
NC = 2
NS = 16
NW = NC * NS
C = 128
K = 5


@functools.partial(jax.jit, static_argnums=(2,))
def _gather_sc(idx, table, nchunk):
    assert nchunk % (2 * K) == 0
    nmac = nchunk // K
    pairs = nmac // 2
    mesh = plsc.VectorSubcoreMesh(core_axis_name="c", subcore_axis_name="s")

    @functools.partial(
        pl.kernel,
        mesh=mesh,
        out_type=jax.ShapeDtypeStruct((NW, nmac, K * C, EMB), jnp.float32),
        scratch_types=[
            pltpu.VMEM((nchunk, C), jnp.int32),
            pltpu.VMEM((K * C, EMB), jnp.float32),
            pltpu.VMEM((K * C, EMB), jnp.float32),
            pltpu.SemaphoreType.DMA,
            pltpu.SemaphoreType.DMA,
            pltpu.SemaphoreType.DMA,
            pltpu.SemaphoreType.DMA,
        ],
        compiler_params=pltpu.CompilerParams(use_tc_tiling_on_sc=False),
    )
    def k(idx_hbm, table_hbm, out_hbm, idx_v, rows0, rows1,
          gsem0, gsem1, ssem0, ssem1):
        wid = lax.axis_index("s") * NC + lax.axis_index("c")
        pltpu.sync_copy(idx_hbm.at[wid], idx_v)

        rows = (rows0, rows1)
        gsems = (gsem0, gsem1)
        ssems = (ssem0, ssem1)

        def pair(p, carry):
            for buf in range(2):
                m = p * 2 + buf
                @pl.when(p > 0)
                def _wait():
                    pltpu.make_async_copy(
                        rows[buf], out_hbm.at[wid, m], ssems[buf]).wait()
                descs = []
                for t in range(K):
                    descs.append(pltpu.async_copy(
                        table_hbm.at[idx_v.at[m * K + t]],
                        rows[buf].at[pl.ds(t * C, C)],
                        gsems[buf]))
                for d in descs:
                    d.wait()
                pltpu.async_copy(rows[buf], out_hbm.at[wid, m], ssems[buf])
            return carry

        lax.fori_loop(0, pairs, pair, 0)
        for buf in range(2):
            pltpu.make_async_copy(
                rows[buf], out_hbm.at[wid, 0], ssems[buf]).wait()

    return k(idx, table)


def kernel(char_ids, table):
    B, L = char_ids.shape
    total = B * L
    assert total % (NW * C) == 0
    nchunk = total // (NW * C)
    idx = char_ids.reshape(NW, nchunk, C)
    out = _gather_sc(idx, table, nchunk)
    return out.reshape(B, L, EMB)


# speedup vs baseline: 4.2591x; 1.0043x over previous
import functools

import jax
import jax.numpy as jnp
from jax import lax
from jax.experimental import pallas as pl
from jax.experimental.pallas import tpu as pltpu
from jax.experimental.pallas import tpu_sc as plsc

EMB = 64
NC = 2
NS = 16
NW = NC * NS
C = 128
NB = 10
D = 5


@functools.partial(jax.jit, static_argnums=(2,))
def _gather_sc(idx, table, nchunk):
    assert nchunk % NB == 0
    ngroup = nchunk // NB
    mesh = plsc.VectorSubcoreMesh(core_axis_name="c", subcore_axis_name="s")

    @functools.partial(
        pl.kernel,
        mesh=mesh,
        out_type=jax.ShapeDtypeStruct((NW, nchunk, C, EMB), jnp.float32),
        scratch_types=(
            [pltpu.VMEM((nchunk, C), jnp.int32),
             pltpu.VMEM((NB, C, EMB), jnp.float32)]
            + [pltpu.SemaphoreType.DMA] * (2 * NB)
        ),
        compiler_params=pltpu.CompilerParams(use_tc_tiling_on_sc=False),
    )
    def k(idx_hbm, table_hbm, out_hbm, idx_v, rows, *sems):
        gsem = sems[:NB]
        ssem = sems[NB:]
        wid = lax.axis_index("s") * NC + lax.axis_index("c")
        pltpu.sync_copy(idx_hbm.at[wid], idx_v)

        def fire_gather(j, b):
            pltpu.async_copy(table_hbm.at[idx_v.at[j]], rows.at[b], gsem[b])

        def wait_gather(b):
            pltpu.make_async_copy(out_hbm.at[wid, 0], rows.at[b],
                                  gsem[b]).wait()

        def fire_scatter(j, b):
            pltpu.async_copy(rows.at[b], out_hbm.at[wid, j], ssem[b])

        def wait_scatter(b):
            pltpu.make_async_copy(rows.at[b], out_hbm.at[wid, 0],
                                  ssem[b]).wait()

        for b in range(NB):
            fire_gather(b, b)
            if b >= D:
                b2 = b - D
                wait_gather(b2)
                fire_scatter(b2, b2)

        def group(g, carry):
            j0 = g * NB
            for b in range(NB):
                j = j0 + b
                wait_scatter(b)
                fire_gather(j, b)
                b2 = (b + NB - D) % NB
                wait_gather(b2)
                fire_scatter(j - D, b2)
            return carry

        lax.fori_loop(1, ngroup, group, 0)

        j0 = (ngroup - 1) * NB
        for b in range(NB - D, NB):
            wait_gather(b)
            fire_scatter(j0 + b, b)
        for b in range(NB):
            wait_scatter(b)

    return k(idx, table)


def kernel(char_ids, table):
    B, L = char_ids.shape
    total = B * L
    assert total % (NW * C) == 0
    nchunk = total // (NW * C)
    idx = char_ids.reshape(NW, nchunk, C)
    out = _gather_sc(idx, table, nchunk)
    return out.reshape(B, L, EMB)
